# baseline (speedup 1.0000x reference)
import jax
import jax.numpy as jnp
from jax.experimental import pallas as pl
from jax.experimental.pallas import tpu as pltpu
from jax.experimental.pallas import tpu_sc as plsc

N = 8192
K = 8192
D = 256
BM = 1024
BK = 2048
NM = N // BM
NK = K // BK
CPB = BK // 128
GW = 128


NH = N // 2


def _dist_argmin_body(x_ref, w_ref, a_ref, wn_ref, idx_ref):
    mi = pl.program_id(0)
    xb = x_ref[pl.ds(mi * BM, BM), :].astype(jnp.bfloat16)
    ab = a_ref[pl.ds(mi * BM, BM), :]
    lane = jax.lax.broadcasted_iota(jnp.int32, (BM, 128), 1)

    acc_v = None
    acc_i = None
    for k in range(NK):
        wb = w_ref[pl.ds(k * BK, BK), :].astype(jnp.bfloat16)
        m = jax.lax.dot_general(
            xb, wb, (((1,), (1,)), ((), ())),
            preferred_element_type=jnp.float32)
        wn = wn_ref[:, pl.ds(k * BK, BK)]
        d = (ab + wn) - 2.0 * m

        rv = jnp.full((BM, 128), jnp.inf, jnp.float32)
        rc = jnp.zeros((BM, 128), jnp.int32)
        for c in range(CPB):
            dc = d[:, c * 128:(c + 1) * 128]
            msk = dc < rv
            rv = jnp.where(msk, dc, rv)
            rc = jnp.where(msk, k * CPB + c, rc)

        rowmin = jnp.min(rv, axis=1, keepdims=True)
        wins = jnp.where(rv == rowmin, rc * 128 + lane,
                         jnp.int32(2**31 - 1))
        wi = jnp.min(wins, axis=1, keepdims=True)

        if acc_v is None:
            acc_v, acc_i = rowmin, wi
        else:
            take = rowmin < acc_v
            acc_v = jnp.where(take, rowmin, acc_v)
            acc_i = jnp.where(take, wi, acc_i)
        acc_v = acc_v.astype(jnp.bfloat16).astype(jnp.float32)

    idx_ref[...] = acc_i[:, 0]


def _argmin_indices(flat, W, a, wn):
    return pl.pallas_call(
        _dist_argmin_body,
        grid=(NH // BM,),
        in_specs=[
            pl.BlockSpec((NH, D), lambda m: (0, 0)),
            pl.BlockSpec((K, D), lambda m: (0, 0)),
            pl.BlockSpec((NH, 1), lambda m: (0, 0)),
            pl.BlockSpec((1, K), lambda m: (0, 0)),
        ],
        out_specs=pl.BlockSpec((BM,), lambda m: (m,)),
        out_shape=jax.ShapeDtypeStruct((NH,), jnp.int32),
        compiler_params=pltpu.CompilerParams(
            dimension_semantics=("arbitrary",)),
    )(flat, W, a, wn)


def _sc_gather(W, idx):
    mesh = plsc.VectorSubcoreMesh(
        core_axis_name="core", subcore_axis_name="subcore")

    @pl.kernel(out_type=jax.ShapeDtypeStruct((NH, D), jnp.float32), mesh=mesh)
    def kern(w_hbm, i_hbm, o_hbm):
        def body(i_vmem, o_vmem):
            pltpu.sync_copy(w_hbm.at[i_vmem.at[0]], o_vmem)

        pltpu.emit_pipeline(
            body,
            grid=(NH // GW,),
            in_specs=[pl.BlockSpec((1, GW), index_map=lambda i: (0, i))],
            out_specs=[pl.BlockSpec((GW, D), index_map=lambda i: (i, 0))],
            core_axis_name=("core", "subcore"),
            dimension_semantics=(pltpu.PARALLEL,),
        )(i_hbm, o_hbm)

    return kern(W, idx.reshape(1, NH))


def kernel(x, W):
    flat = x.reshape(-1, D)
    a = jnp.sum(flat ** 2, axis=1, keepdims=True)
    wn = jnp.sum(W ** 2, axis=1).reshape(1, K)
    idx0 = _argmin_indices(flat[:NH], W, a[:NH], wn)
    q0 = _sc_gather(W, idx0)
    idx1 = _argmin_indices(flat[NH:], W, a[NH:], wn)
    q1 = _sc_gather(W, idx1)
    quant = jnp.concatenate([q0, q1], axis=0)
    idx = jnp.concatenate([idx0, idx1], axis=0)
    return quant.reshape(x.shape), idx[:, None]


# speedup vs baseline: 1.1486x; 1.1486x over previous
import jax
import jax.numpy as jnp
from jax.experimental import pallas as pl
from jax.experimental.pallas import tpu as pltpu
from jax.experimental.pallas import tpu_sc as plsc

N = 8192
K = 8192
D = 256
BM = 1024
BK = 2048
NM = N // BM
NK = K // BK
CPB = BK // 128
GW = 128


def _dist_argmin_body(x_ref, w_ref, a_ref, wn_ref, idx_ref):
    mi = pl.program_id(0)
    xb = x_ref[pl.ds(mi * BM, BM), :].astype(jnp.bfloat16)
    ab = a_ref[pl.ds(mi * BM, BM), :]
    lane = jax.lax.broadcasted_iota(jnp.int32, (BM, 128), 1)

    acc_v = None
    acc_i = None
    for k in range(NK):
        wb = w_ref[pl.ds(k * BK, BK), :].astype(jnp.bfloat16)
        m = jax.lax.dot_general(
            xb, wb, (((1,), (1,)), ((), ())),
            preferred_element_type=jnp.float32)
        wn = wn_ref[:, pl.ds(k * BK, BK)]
        d = (ab + wn) - 2.0 * m

        rv = jnp.full((BM, 128), jnp.inf, jnp.float32)
        rc = jnp.zeros((BM, 128), jnp.int32)
        for c in range(CPB):
            dc = d[:, c * 128:(c + 1) * 128]
            msk = dc < rv
            rv = jnp.where(msk, dc, rv)
            rc = jnp.where(msk, k * CPB + c, rc)

        rowmin = jnp.min(rv, axis=1, keepdims=True)
        wins = jnp.where(rv == rowmin, rc * 128 + lane,
                         jnp.int32(2**31 - 1))
        wi = jnp.min(wins, axis=1, keepdims=True)

        if acc_v is None:
            acc_v, acc_i = rowmin, wi
        else:
            take = rowmin < acc_v
            acc_v = jnp.where(take, rowmin, acc_v)
            acc_i = jnp.where(take, wi, acc_i)
        acc_v = acc_v.astype(jnp.bfloat16).astype(jnp.float32)

    idx_ref[...] = acc_i[:, 0]


def _argmin_indices(flat, W, a, wn):
    return pl.pallas_call(
        _dist_argmin_body,
        grid=(NM,),
        in_specs=[
            pl.BlockSpec((N, D), lambda m: (0, 0)),
            pl.BlockSpec((K, D), lambda m: (0, 0)),
            pl.BlockSpec((N, 1), lambda m: (0, 0)),
            pl.BlockSpec((1, K), lambda m: (0, 0)),
        ],
        out_specs=pl.BlockSpec((BM,), lambda m: (m,)),
        out_shape=jax.ShapeDtypeStruct((N,), jnp.int32),
        compiler_params=pltpu.CompilerParams(
            dimension_semantics=("arbitrary",)),
    )(flat, W, a, wn)


def _sc_gather(W, idx):
    mesh = plsc.VectorSubcoreMesh(
        core_axis_name="core", subcore_axis_name="subcore")

    @pl.kernel(out_type=jax.ShapeDtypeStruct((N, D), jnp.float32), mesh=mesh)
    def kern(w_hbm, i_hbm, o_hbm):
        def body(i_vmem, o_vmem):
            pltpu.sync_copy(w_hbm.at[i_vmem.at[0]], o_vmem)

        pltpu.emit_pipeline(
            body,
            grid=(N // GW,),
            in_specs=[pl.BlockSpec((1, GW), index_map=lambda i: (0, i))],
            out_specs=[pl.BlockSpec((GW, D), index_map=lambda i: (i, 0))],
            core_axis_name=("core", "subcore"),
            dimension_semantics=(pltpu.PARALLEL,),
        )(i_hbm, o_hbm)

    return kern(W, idx.reshape(1, N))


def kernel(x, W):
    flat = x.reshape(-1, D)
    a = jnp.sum(flat ** 2, axis=1, keepdims=True)
    wn = jnp.sum(W ** 2, axis=1)
    idx = _argmin_indices(flat, W, a, wn.reshape(1, K))
    quant = _sc_gather(W, idx)
    return quant.reshape(x.shape), idx[:, None]
